# Initial kernel scaffold; baseline (speedup 1.0000x reference)
import jax
import jax.numpy as jnp
from jax.experimental import pallas as pl


def kernel(x):
    raise NotImplementedError("write your pallas kernel here")


# speedup vs baseline: 4.5526x; 4.5526x over previous
import functools

import jax
import jax.numpy as jnp
from jax import lax
from jax.experimental import pallas as pl
from jax.experimental.pallas import tpu as pltpu
from jax.experimental.pallas import tpu_sc as plsc

N_ROWS = 16384
N_COLS = 832
N_FIELDS_K = 26
FIELD = 32
LANES = 16

NW = 32
ROWS_PER_W = N_ROWS // NW
CHUNK = 64
N_CHUNKS = ROWS_PER_W // CHUNK


def _sc_body(x_hbm, out_hbm, buf):
    wid = lax.axis_index("s") * 2 + lax.axis_index("c")
    ji = lax.iota(jnp.int32, LANES)

    def chunk_body(k, _):
        base = wid * ROWS_PER_W + k * CHUNK
        pltpu.sync_copy(x_hbm.at[pl.ds(base, CHUNK)], buf)

        def row_body(r, _):
            for f in range(N_FIELDS_K):
                c = f * FIELD
                v0 = buf[r, pl.ds(c, LANES)]
                v1 = buf[r, pl.ds(c + LANES, LANES)]
                vm = jnp.maximum(v0, v1)
                for d in (1, 2, 4, 8):
                    vm = jnp.maximum(
                        vm, vm.at[ji ^ d].get(mode="promise_in_bounds"))
                k0 = jnp.where(v0 == vm, ji, 64)
                k1 = jnp.where(v1 == vm, ji + LANES, 64)
                km = jnp.minimum(k0, k1)
                for d in (1, 2, 4, 8):
                    km = jnp.minimum(
                        km, km.at[ji ^ d].get(mode="promise_in_bounds"))
                buf[r, pl.ds(c, LANES)] = jnp.where(
                    ji == km, 1.0, 0.0).astype(jnp.float32)
                buf[r, pl.ds(c + LANES, LANES)] = jnp.where(
                    ji + LANES == km, 1.0, 0.0).astype(jnp.float32)
            return 0

        lax.fori_loop(0, CHUNK, row_body, 0)
        pltpu.sync_copy(buf, out_hbm.at[pl.ds(base, CHUNK)])
        return 0

    lax.fori_loop(0, N_CHUNKS, chunk_body, 0)


@jax.jit
def kernel(x):
    mesh = plsc.VectorSubcoreMesh(core_axis_name="c", subcore_axis_name="s")
    f = functools.partial(
        pl.kernel,
        mesh=mesh,
        out_type=jax.ShapeDtypeStruct((N_ROWS, N_COLS), jnp.float32),
        scratch_types=[pltpu.VMEM((CHUNK, N_COLS), jnp.float32)],
    )(_sc_body)
    return f(x)


# speedup vs baseline: 4.5685x; 1.0035x over previous
import functools

import jax
import jax.numpy as jnp
from jax import lax
from jax.experimental import pallas as pl
from jax.experimental.pallas import tpu as pltpu
from jax.experimental.pallas import tpu_sc as plsc

N_ROWS = 16384
N_COLS = 832
N_FIELDS_K = 26
FIELD = 32
LANES = 16

NW = 32
ROWS_PER_W = N_ROWS // NW
CHUNK = 64
N_CHUNKS = ROWS_PER_W // CHUNK


def _sc_body(x_hbm, out_hbm, buf):
    wid = lax.axis_index("s") * 2 + lax.axis_index("c")
    ji = lax.iota(jnp.int32, LANES)

    def chunk_body(k, _):
        base = wid * ROWS_PER_W + k * CHUNK
        pltpu.sync_copy(x_hbm.at[pl.ds(base, CHUNK)], buf)

        def row_body(r, _):
            for f in range(N_FIELDS_K):
                c = f * FIELD
                v0 = buf[r, pl.ds(c, LANES)]
                v1 = buf[r, pl.ds(c + LANES, LANES)]
                vm = jnp.maximum(v0, v1)
                for d in (1, 2, 4, 8):
                    vm = jnp.maximum(
                        vm, vm.at[ji ^ d].get(mode="promise_in_bounds"))
                k0 = jnp.where(v0 == vm, ji, 64)
                k1 = jnp.where(v1 == vm, ji + LANES, 64)
                km = jnp.minimum(k0, k1)
                for d in (1, 2, 4, 8):
                    km = jnp.minimum(
                        km, km.at[ji ^ d].get(mode="promise_in_bounds"))
                buf[r, pl.ds(c, LANES)] = jnp.where(
                    ji == km, 1.0, 0.0).astype(jnp.float32)
                buf[r, pl.ds(c + LANES, LANES)] = jnp.where(
                    ji + LANES == km, 1.0, 0.0).astype(jnp.float32)
            return 0

        lax.fori_loop(0, CHUNK, row_body, 0)
        pltpu.sync_copy(buf, out_hbm.at[pl.ds(base, CHUNK)])
        return 0

    lax.fori_loop(0, N_CHUNKS, chunk_body, 0)


@jax.jit
def kernel(x):
    mesh = plsc.VectorSubcoreMesh(core_axis_name="c", subcore_axis_name="s")
    f = functools.partial(
        pl.kernel,
        mesh=mesh,
        out_type=jax.ShapeDtypeStruct((N_ROWS, N_COLS), jnp.float32),
        scratch_types=[pltpu.VMEM((CHUNK, N_COLS), jnp.float32)],
        compiler_params=pltpu.CompilerParams(use_tc_tiling_on_sc=True),
    )(_sc_body)
    return f(x)


# speedup vs baseline: 5.3817x; 1.1780x over previous
import functools

import jax
import jax.numpy as jnp
from jax import lax
from jax.experimental import pallas as pl
from jax.experimental.pallas import tpu as pltpu
from jax.experimental.pallas import tpu_sc as plsc

N_ROWS = 16384
N_COLS = 832
N_FIELDS_K = 26
FIELD = 32
LANES = 16

NW = 32
ROWS_PER_W = N_ROWS // NW
CHUNK = 64
N_CHUNKS = ROWS_PER_W // CHUNK


def _sc_body(x_hbm, out_hbm, buf):
    wid = lax.axis_index("s") * 2 + lax.axis_index("c")
    ji = lax.iota(jnp.int32, LANES)

    def chunk_body(k, _):
        base = wid * ROWS_PER_W + k * CHUNK
        pltpu.sync_copy(x_hbm.at[pl.ds(base, CHUNK)], buf)

        def row_body(r, _):
            for f in range(N_FIELDS_K):
                c = f * FIELD
                v0 = buf[r, pl.ds(c, LANES)]
                v1 = buf[r, pl.ds(c + LANES, LANES)]
                m = jnp.max(jnp.maximum(v0, v1))
                eq0 = v0 == m
                eq1 = v1 == m
                n0 = plsc.all_reduce_population_count(eq0)
                f0 = plsc.all_reduce_ffs(eq0)
                f1 = plsc.all_reduce_ffs(eq1)
                first = jnp.where(n0 > 0, f0, f1 + LANES)
                buf[r, pl.ds(c, LANES)] = jnp.where(
                    ji == first, 1.0, 0.0).astype(jnp.float32)
                buf[r, pl.ds(c + LANES, LANES)] = jnp.where(
                    ji == first - LANES, 1.0, 0.0).astype(jnp.float32)
            return 0

        lax.fori_loop(0, CHUNK, row_body, 0)
        pltpu.sync_copy(buf, out_hbm.at[pl.ds(base, CHUNK)])
        return 0

    lax.fori_loop(0, N_CHUNKS, chunk_body, 0)


@jax.jit
def kernel(x):
    mesh = plsc.VectorSubcoreMesh(core_axis_name="c", subcore_axis_name="s")
    f = functools.partial(
        pl.kernel,
        mesh=mesh,
        out_type=jax.ShapeDtypeStruct((N_ROWS, N_COLS), jnp.float32),
        scratch_types=[pltpu.VMEM((CHUNK, N_COLS), jnp.float32)],
        compiler_params=pltpu.CompilerParams(needs_layout_passes=False),
    )(_sc_body)
    return f(x)
